# Initial kernel scaffold; baseline (speedup 1.0000x reference)
import jax
import jax.numpy as jnp
from jax.experimental import pallas as pl


def kernel(v, l_filter, r_filter):
    raise NotImplementedError("write your pallas kernel here")


# speedup vs baseline: 123.6564x; 123.6564x over previous
import functools

import jax
import jax.numpy as jnp
from jax import lax
from jax.experimental import pallas as pl
from jax.experimental.pallas import tpu as pltpu
from jax.experimental.pallas import tpu_sc as plsc

L_ORDER = 10
R_ORDER = 2
NC, NS = 2, 16
LANES = 16


def _dfsmn_body(v_hbm, l_hbm, r_hbm, out_hbm, buf, lbuf, rbuf):
    BH, T, _, DH = v_hbm.shape
    G = DH // LANES
    wid = lax.axis_index("s") * NC + lax.axis_index("c")
    bh = wid // 2
    dh = wid % 2

    pltpu.sync_copy(v_hbm.at[bh, :, dh], buf.at[pl.ds(0, T)])
    pltpu.sync_copy(l_hbm.at[:, dh], lbuf)
    pltpu.sync_copy(r_hbm.at[:, dh], rbuf)

    zero = jnp.zeros((LANES,), jnp.float32)
    for g in range(G):
        gs = pl.ds(g * LANES, LANES)
        buf[T, gs] = zero
        buf[T + 1, gs] = zero

    for g in range(G):
        gs = pl.ds(g * LANES, LANES)
        c0 = lbuf[0, gs] + 1.0
        l1, l2, l3, l4, l5, l6, l7, l8, l9 = (
            lbuf[k, gs] for k in range(1, L_ORDER))
        r1 = rbuf[0, gs]
        r2 = rbuf[1, gs]

        def step(t, carry, gs=gs, c0=c0, l1=l1, l2=l2, l3=l3, l4=l4, l5=l5,
                 l6=l6, l7=l7, l8=l8, l9=l9, r1=r1, r2=r2):
            d1, d2, d3, d4, d5, d6, d7, d8, d9, e1, e2 = carry
            w = buf[t + 2, gs]
            y = (e1 + r2 * w) + d1
            buf[t, gs] = y
            return (d2 + l1 * y, d3 + l2 * y, d4 + l3 * y, d5 + l4 * y,
                    d6 + l5 * y, d7 + l6 * y, d8 + l7 * y, d9 + l8 * y,
                    l9 * y, e2 + r1 * w, c0 * w)

        v0 = buf[0, gs]
        v1 = buf[1, gs]
        init = (zero,) * 9 + (c0 * v0 + r1 * v1, c0 * v1)
        lax.fori_loop(0, T, step, init, unroll=4)

    pltpu.sync_copy(buf.at[pl.ds(0, T)], out_hbm.at[bh, :, dh])


@functools.partial(jax.jit, static_argnames=())
def kernel(v, l_filter, r_filter):
    B, H, T, D = v.shape
    DH = D // 2
    v2 = v.reshape(B * H, T, 2, DH)
    l2 = l_filter.reshape(L_ORDER, 2, DH)
    r2 = r_filter.reshape(R_ORDER, 2, DH)
    run = pl.kernel(
        _dfsmn_body,
        out_type=jax.ShapeDtypeStruct((B * H, T, 2, DH), jnp.float32),
        mesh=plsc.VectorSubcoreMesh(
            core_axis_name="c", subcore_axis_name="s",
            num_cores=NC, num_subcores=NS),
        scratch_types=[
            pltpu.VMEM((T + 2, DH), jnp.float32),
            pltpu.VMEM((L_ORDER, DH), jnp.float32),
            pltpu.VMEM((R_ORDER, DH), jnp.float32),
        ],
    )
    out = run(v2, l2, r2)
    return out.reshape(B, H, T, D)
